# baseline (speedup 1.0000x reference)
import functools

import jax
import jax.numpy as jnp
from jax import lax
from jax.experimental import pallas as pl
from jax.experimental.pallas import tpu as pltpu
from jax.experimental.pallas import tpu_sc as plsc

N, D, B = 100000, 128, 1024

GR = 20000


def _gate_body(x_ref, w_ref, b_ref, out_ref):
    xb = x_ref[...]
    z = lax.dot_general(xb, w_ref[...], (((1,), (0,)), ((), ())),
                        preferred_element_type=jnp.float32) + b_ref[0]
    out_ref[...] = xb * jax.nn.sigmoid(z)


def _gate_pass(x, W, b):
    return pl.pallas_call(
        _gate_body,
        grid=(N // GR,),
        in_specs=[
            pl.BlockSpec((GR, D), lambda i: (i, 0)),
            pl.BlockSpec((D, 1), lambda i: (0, 0)),
            pl.BlockSpec((1,), lambda i: (0,)),
        ],
        out_specs=pl.BlockSpec((GR, D), lambda i: (i, 0)),
        out_shape=jax.ShapeDtypeStruct((N, D), jnp.float32),
    )(x, W, b)


NC, NS = 2, 16
NW = NC * NS
NB = 4
CH = 80
NCHUNK = N // CH
CPT = NCHUNK // NW
EXTRA = NCHUNK - CPT * NW
ZR = B // NS
SLOTS = ((CPT + 3) // NB + 1) * NB


def _sc_body(w_hbm, ids_hbm, p0_hbm, p1_hbm,
             rb0, rb1, rb2, rb3, ib0, ib1, ib2, ib3, acc,
             sr0, sr1, sr2, sr3, si0, si1, si2, si3,
             ss0, ss1, ss2, ss3):
    cid = lax.axis_index("c")
    sid = lax.axis_index("s")
    w = cid * NS + sid
    base = w * CPT + jnp.minimum(w, EXTRA)
    n = CPT + (w < EXTRA).astype(jnp.int32)

    rbufs, ibufs = (rb0, rb1, rb2, rb3), (ib0, ib1, ib2, ib3)
    srs, sis, sss = (sr0, sr1, sr2, sr3), (si0, si1, si2, si3), \
        (ss0, ss1, ss2, ss3)

    def issue(s, b):
        @pl.when((s >= 0) & (s < n))
        def _():
            chunk = base + s
            pltpu.async_copy(w_hbm.at[pl.ds(chunk * CH, CH), :],
                             rbufs[b], srs[b])
            pltpu.async_copy(ids_hbm.at[pl.ds(chunk * CH, CH)],
                             ibufs[b], sis[b])

    def _zrow(r, carry):
        for j in range(D // 16):
            rb0[r, pl.ds(j * 16, 16)] = jnp.zeros((16,), jnp.float32)
        return carry
    lax.fori_loop(0, ZR, _zrow, None)
    pltpu.sync_copy(rb0.at[pl.ds(0, ZR)], acc.at[pl.ds(sid * ZR, ZR)])
    plsc.subcore_barrier()

    issue(0, 0)
    issue(1, 1)

    @pl.loop(0, SLOTS, step=NB)
    def _slot(o):
        for b in range(NB):
            s = o + b
            sp = s - 2
            bp = (b + 2) % NB

            @pl.when((sp >= 0) & (sp < n))
            def _():
                pltpu.make_async_copy(rbufs[bp], acc.at[ibufs[bp]],
                                      sss[bp]).wait()
            issue(s + 2, bp)

            @pl.when(s < n)
            def _():
                chunk = base + s
                pltpu.make_async_copy(w_hbm.at[pl.ds(chunk * CH, CH), :],
                                      rbufs[b], srs[b]).wait()
                pltpu.make_async_copy(ids_hbm.at[pl.ds(chunk * CH, CH)],
                                      ibufs[b], sis[b]).wait()
                pltpu.async_copy(rbufs[b], acc.at[ibufs[b]], sss[b],
                                 add=True)

    plsc.subcore_barrier()

    @pl.when(cid == 0)
    def _():
        pltpu.sync_copy(acc.at[pl.ds(sid * ZR, ZR)],
                        p0_hbm.at[pl.ds(sid * ZR, ZR)])

    @pl.when(cid == 1)
    def _():
        pltpu.sync_copy(acc.at[pl.ds(sid * ZR, ZR)],
                        p1_hbm.at[pl.ds(sid * ZR, ZR)])


def _sc_scatter(weighted, batch_id):
    mesh = plsc.VectorSubcoreMesh(core_axis_name="c", subcore_axis_name="s",
                                  num_cores=NC, num_subcores=NS)
    f = pl.kernel(
        _sc_body,
        out_type=(jax.ShapeDtypeStruct((B, D), jnp.float32),
                  jax.ShapeDtypeStruct((B, D), jnp.float32)),
        mesh=mesh,
        scratch_types=(
            [pltpu.VMEM((CH, D), jnp.float32) for _ in range(NB)]
            + [pltpu.VMEM((CH,), jnp.int32) for _ in range(NB)]
            + [pltpu.VMEM_SHARED((B, D), jnp.float32)]
            + [pltpu.SemaphoreType.DMA for _ in range(3 * NB)]
        ),
    )
    return f(weighted, batch_id)


def _combine_body(p0_ref, p1_ref, out_ref):
    out_ref[...] = p0_ref[...] + p1_ref[...]


def _combine(p0, p1):
    return pl.pallas_call(
        _combine_body,
        in_specs=[pl.BlockSpec((B, D), lambda: (0, 0)),
                  pl.BlockSpec((B, D), lambda: (0, 0))],
        out_specs=pl.BlockSpec((B, D), lambda: (0, 0)),
        out_shape=jax.ShapeDtypeStruct((B, D), jnp.float32),
    )(p0, p1)


def kernel(x, batch_id, batch_size, W, b):
    weighted = _gate_pass(x, W, b)
    p0, p1 = _sc_scatter(weighted, batch_id)
    out = _combine(p0, p1)
    return out + jnp.asarray(batch_size - B, dtype=out.dtype)


# speedup vs baseline: 1.0225x; 1.0225x over previous
import functools

import jax
import jax.numpy as jnp
from jax import lax
from jax.experimental import pallas as pl
from jax.experimental.pallas import tpu as pltpu
from jax.experimental.pallas import tpu_sc as plsc

N, D, B = 100000, 128, 1024

GR = 10000


def _gate_body(x_ref, w_ref, b_ref, out_ref):
    xb = x_ref[...]
    z = lax.dot_general(xb, w_ref[...], (((1,), (0,)), ((), ())),
                        preferred_element_type=jnp.float32) + b_ref[0]
    out_ref[...] = xb * jax.nn.sigmoid(z)


def _gate_pass(x, W, b):
    return pl.pallas_call(
        _gate_body,
        grid=(N // GR,),
        in_specs=[
            pl.BlockSpec((GR, D), lambda i: (i, 0)),
            pl.BlockSpec((D, 1), lambda i: (0, 0)),
            pl.BlockSpec((1,), lambda i: (0,)),
        ],
        out_specs=pl.BlockSpec((GR, D), lambda i: (i, 0)),
        out_shape=jax.ShapeDtypeStruct((N, D), jnp.float32),
    )(x, W, b)


NC, NS = 2, 16
NW = NC * NS
NB = 4
CH = 160
SUB = CH // 2
NCHUNK = N // CH
CPT = NCHUNK // NW
EXTRA = NCHUNK - CPT * NW
ZR = B // NS
SLOTS = ((CPT + 3) // NB + 1) * NB


def _sc_body(w_hbm, ids_hbm, p0_hbm, p1_hbm,
             rb0, rb1, rb2, rb3, ib0, ib1, ib2, ib3, acc,
             sr0, sr1, sr2, sr3, si0, si1, si2, si3,
             ss0, ss1, ss2, ss3):
    cid = lax.axis_index("c")
    sid = lax.axis_index("s")
    w = cid * NS + sid
    base = w * CPT + jnp.minimum(w, EXTRA)
    n = CPT + (w < EXTRA).astype(jnp.int32)

    rbufs, ibufs = (rb0, rb1, rb2, rb3), (ib0, ib1, ib2, ib3)
    srs, sis, sss = (sr0, sr1, sr2, sr3), (si0, si1, si2, si3), \
        (ss0, ss1, ss2, ss3)

    def issue(s, b):
        @pl.when((s >= 0) & (s < n))
        def _():
            chunk = base + s
            pltpu.async_copy(w_hbm.at[pl.ds(chunk * CH, CH), :],
                             rbufs[b], srs[b])
            for j in range(2):
                pltpu.async_copy(
                    ids_hbm.at[pl.ds(chunk * CH + j * SUB, SUB)],
                    ibufs[b].at[j], sis[b])

    def _zrow(r, carry):
        for j in range(D // 16):
            rb0[r, pl.ds(j * 16, 16)] = jnp.zeros((16,), jnp.float32)
        return carry
    lax.fori_loop(0, ZR, _zrow, None)
    pltpu.sync_copy(rb0.at[pl.ds(0, ZR)], acc.at[pl.ds(sid * ZR, ZR)])
    plsc.subcore_barrier()

    issue(0, 0)
    issue(1, 1)

    @pl.loop(0, SLOTS, step=NB)
    def _slot(o):
        for b in range(NB):
            s = o + b
            sp = s - 2
            bp = (b + 2) % NB

            @pl.when((sp >= 0) & (sp < n))
            def _():
                for j in range(2):
                    pltpu.make_async_copy(
                        rbufs[bp].at[pl.ds(j * SUB, SUB), :],
                        acc.at[ibufs[bp].at[j]], sss[bp]).wait()
            issue(s + 2, bp)

            @pl.when(s < n)
            def _():
                chunk = base + s
                pltpu.make_async_copy(w_hbm.at[pl.ds(chunk * CH, CH), :],
                                      rbufs[b], srs[b]).wait()
                for j in range(2):
                    pltpu.make_async_copy(
                        ids_hbm.at[pl.ds(chunk * CH + j * SUB, SUB)],
                        ibufs[b].at[j], sis[b]).wait()
                for j in range(2):
                    pltpu.async_copy(rbufs[b].at[pl.ds(j * SUB, SUB), :],
                                     acc.at[ibufs[b].at[j]], sss[b],
                                     add=True)

    plsc.subcore_barrier()

    @pl.when(cid == 0)
    def _():
        pltpu.sync_copy(acc.at[pl.ds(sid * ZR, ZR)],
                        p0_hbm.at[pl.ds(sid * ZR, ZR)])

    @pl.when(cid == 1)
    def _():
        pltpu.sync_copy(acc.at[pl.ds(sid * ZR, ZR)],
                        p1_hbm.at[pl.ds(sid * ZR, ZR)])


def _sc_scatter(weighted, batch_id):
    mesh = plsc.VectorSubcoreMesh(core_axis_name="c", subcore_axis_name="s",
                                  num_cores=NC, num_subcores=NS)
    f = pl.kernel(
        _sc_body,
        out_type=(jax.ShapeDtypeStruct((B, D), jnp.float32),
                  jax.ShapeDtypeStruct((B, D), jnp.float32)),
        mesh=mesh,
        scratch_types=(
            [pltpu.VMEM((CH, D), jnp.float32) for _ in range(NB)]
            + [pltpu.VMEM((2, SUB), jnp.int32) for _ in range(NB)]
            + [pltpu.VMEM_SHARED((B, D), jnp.float32)]
            + [pltpu.SemaphoreType.DMA for _ in range(3 * NB)]
        ),
    )
    return f(weighted, batch_id)


def _combine_body(p0_ref, p1_ref, out_ref):
    out_ref[...] = p0_ref[...] + p1_ref[...]


def _combine(p0, p1):
    return pl.pallas_call(
        _combine_body,
        in_specs=[pl.BlockSpec((B, D), lambda: (0, 0)),
                  pl.BlockSpec((B, D), lambda: (0, 0))],
        out_specs=pl.BlockSpec((B, D), lambda: (0, 0)),
        out_shape=jax.ShapeDtypeStruct((B, D), jnp.float32),
    )(p0, p1)


def kernel(x, batch_id, batch_size, W, b):
    weighted = _gate_pass(x, W, b)
    p0, p1 = _sc_scatter(weighted, batch_id)
    out = _combine(p0, p1)
    return out + jnp.asarray(batch_size - B, dtype=out.dtype)
